# baseline (speedup 1.0000x reference)
import jax
import jax.numpy as jnp
from jax import lax
from jax.experimental import pallas as pl
from jax.experimental.pallas import tpu as pltpu
from jax.experimental.pallas import tpu_sc as plsc

VOCAB = 100000
MAXLEN = 200
EMBED = 64
BATCH = 1024

NC = 2
NS = 16
NW = NC * NS
LANES = 16

BBLK = 128
NCBLK = BATCH // BBLK
N_UNITS = MAXLEN * NCBLK
U_PER_W = N_UNITS // NW
Q = EMBED // LANES
NB = 4
USLAB = EMBED * BBLK
LSLAB = EMBED * BATCH


def _emb_kernel(idxT_hbm, tok_hbm, pos_hbm, out_hbm,
                idx_v, g_v, u_v, pos_v, *sems):
    semg = sems[:NB]
    semo = sems[NB:]
    wid = lax.axis_index("s") * NC + lax.axis_index("c")
    u0 = wid * U_PER_W

    pltpu.sync_copy(pos_hbm, pos_v)

    iota = lax.iota(jnp.int32, LANES)
    dbase = [(q * LANES + iota) * BBLK for q in range(Q)]

    def unit_lc(u):
        gu = u0 + u
        return gu // NCBLK, gu % NCBLK

    def start_gather(u):
        b = u % NB
        l, c = unit_lc(u)
        pltpu.sync_copy(idxT_hbm.at[l, pl.ds(c * BBLK, BBLK)], idx_v.at[b])
        return pltpu.async_copy(tok_hbm.at[idx_v.at[b]], g_v.at[b], semg[b])

    def start_out(u):
        b = u % NB
        l, c = unit_lc(u)
        base = l * LSLAB + c * BBLK * 8
        return [
            pltpu.async_copy(u_v.at[b, pl.ds(a * BBLK * 8, BBLK * 8)],
                             out_hbm.at[pl.ds(base + a * BBLK * EMBED, BBLK * 8)],
                             semo[b])
            for a in range(8)
        ]

    pending_g = {0: start_gather(0)}
    pending_o = {}
    for u in range(U_PER_W):
        b = u % NB
        l, c = unit_lc(u)
        nxt = u + 1
        if nxt < U_PER_W:
            if nxt - NB >= 0:
                for d in pending_o.pop(nxt - NB):
                    d.wait()
            pending_g[nxt] = start_gather(nxt)
        pending_g.pop(u).wait()

        pq = [pos_v[l, pl.ds(q * LANES, LANES)] for q in range(Q)]

        def t_body(t, car, b=b, pq=pq):
            tvec = jnp.zeros((LANES,), jnp.int32) + t
            for q in range(Q):
                val = g_v[b, t, pl.ds(q * LANES, LANES)] + pq[q]
                plsc.store_scatter(u_v.at[b], [dbase[q] + tvec], val)
            return car

        lax.fori_loop(0, BBLK, t_body, 0)
        pending_o[u] = start_out(u)

    for u in sorted(pending_o):
        for d in pending_o.pop(u):
            d.wait()


@jax.jit
def _run(idxT, token_table, pos_table):
    mesh = plsc.VectorSubcoreMesh(core_axis_name="c", subcore_axis_name="s")
    f = pl.kernel(
        _emb_kernel,
        out_type=jax.ShapeDtypeStruct((MAXLEN * LSLAB,), jnp.float32),
        mesh=mesh,
        scratch_types=[
            pltpu.VMEM((NB, BBLK), jnp.int32),
            pltpu.VMEM((NB, BBLK, EMBED), jnp.float32),
            pltpu.VMEM((NB, USLAB), jnp.float32),
            pltpu.VMEM((MAXLEN, EMBED), jnp.float32),
        ] + [pltpu.SemaphoreType.DMA] * (2 * NB),
        compiler_params=pltpu.CompilerParams(use_tc_tiling_on_sc=False,
                                             needs_layout_passes=False),
    )
    return f(idxT, token_table, pos_table)


def kernel(inputs, token_table, pos_table):
    idxT = inputs.astype(jnp.int32).T
    flat = _run(idxT, token_table, pos_table)
    out5 = flat.reshape(MAXLEN, 8, NCBLK, 8, BBLK)
    return out5.transpose(2, 4, 0, 1, 3).reshape(BATCH, MAXLEN, EMBED)


# speedup vs baseline: 1.6551x; 1.6551x over previous
import jax
import jax.numpy as jnp
from jax import lax
from jax.experimental import pallas as pl
from jax.experimental.pallas import tpu as pltpu
from jax.experimental.pallas import tpu_sc as plsc

VOCAB = 100000
MAXLEN = 200
EMBED = 64
BATCH = 1024

NC = 2
NS = 16
NW = NC * NS
LANES = 16

BBLK = 128
NCBLK = BATCH // BBLK
N_UNITS = MAXLEN * NCBLK
U_PER_W = N_UNITS // NW
Q = EMBED // LANES
NB = 5
NROUNDS = U_PER_W // NB
UPAD = BBLK + 8


def _emb_kernel(idx_hbm, tok_hbm, pos_hbm, out_hbm,
                idx_v, g_v, u_v, pos_v, *sems):
    semg = sems[:NB]
    semo = sems[NB:]
    wid = lax.axis_index("s") * NC + lax.axis_index("c")
    u0 = wid * U_PER_W

    pltpu.sync_copy(pos_hbm, pos_v)

    iota = lax.iota(jnp.int32, LANES)
    avecs = [(q * LANES + iota) // 8 for q in range(Q)]
    rvecs = [(q * LANES + iota) % 8 for q in range(Q)]

    def unit_lc(u):
        gu = u0 + u
        return gu // NCBLK, gu % NCBLK

    def idx_gather_start(u, j):
        l, c = unit_lc(u)
        pltpu.sync_copy(idx_hbm.at[pl.ds(l * BATCH + c * BBLK, BBLK)],
                        idx_v.at[j])
        pltpu.async_copy(tok_hbm.at[idx_v.at[j]], g_v.at[j], semg[j])

    def gather_wait(j):
        pltpu.make_async_copy(tok_hbm.at[idx_v.at[j]], g_v.at[j],
                              semg[j]).wait()

    def out_refs(u, j):
        l, c = unit_lc(u)
        return u_v.at[j, :, :, pl.ds(0, BBLK)], out_hbm.at[l, :, c]

    for j in range(NB):
        idx_gather_start(j, j)

    def round_body(r, car):
        for j in range(NB):
            u = r * NB + j
            gather_wait(j)

            @pl.when(r > 0)
            def _(u=u, j=j):
                src, dst = out_refs(u - NB, j)
                pltpu.make_async_copy(src, dst, semo[j]).wait()

            l, c = unit_lc(u)
            pq = [pos_v[l, pl.ds(q * LANES, LANES)] for q in range(Q)]

            def t_body(t, car2, j=j, pq=pq):
                tvec = jnp.zeros((LANES,), jnp.int32) + t
                for q in range(Q):
                    val = g_v[j, t, pl.ds(q * LANES, LANES)] + pq[q]
                    plsc.store_scatter(u_v.at[j], [avecs[q], rvecs[q], tvec],
                                       val)
                return car2

            lax.fori_loop(0, BBLK, t_body, 0)
            src, dst = out_refs(u, j)
            pltpu.async_copy(src, dst, semo[j])

            @pl.when(r < NROUNDS - 1)
            def _(u=u, j=j):
                idx_gather_start(u + NB, j)
        return car

    lax.fori_loop(0, NROUNDS, round_body, 0)

    for j in range(NB):
        src, dst = out_refs(U_PER_W - NB + j, j)
        pltpu.make_async_copy(src, dst, semo[j]).wait()


@jax.jit
def _run(idx_flat, token_table, pos_table):
    mesh = plsc.VectorSubcoreMesh(core_axis_name="c", subcore_axis_name="s")
    f = pl.kernel(
        _emb_kernel,
        out_type=jax.ShapeDtypeStruct((MAXLEN, 8, NCBLK, 8, BBLK), jnp.float32),
        mesh=mesh,
        scratch_types=[
            pltpu.VMEM((NB, BBLK), jnp.int32),
            pltpu.VMEM((NB, BBLK, EMBED), jnp.float32),
            pltpu.VMEM((NB, 8, 8, UPAD), jnp.float32),
            pltpu.VMEM((MAXLEN, EMBED), jnp.float32),
        ] + [pltpu.SemaphoreType.DMA] * (2 * NB),
        compiler_params=pltpu.CompilerParams(use_tc_tiling_on_sc=False,
                                             needs_layout_passes=False),
    )
    return f(idx_flat, token_table, pos_table)


def kernel(inputs, token_table, pos_table):
    idx_flat = inputs.astype(jnp.int32).T.reshape(-1)
    out5 = _run(idx_flat, token_table, pos_table)
    return out5.transpose(2, 4, 0, 1, 3).reshape(BATCH, MAXLEN, EMBED)


# speedup vs baseline: 2.5630x; 1.5485x over previous
import jax
import jax.numpy as jnp
from jax import lax
from jax.experimental import pallas as pl
from jax.experimental.pallas import tpu as pltpu
from jax.experimental.pallas import tpu_sc as plsc

VOCAB = 100000
MAXLEN = 200
EMBED = 64
BATCH = 1024

NC = 2
NS = 16
NW = NC * NS
LANES = 16

BBLK = 128
NCBLK = BATCH // BBLK
N_UNITS = MAXLEN * NCBLK
U_PER_W = N_UNITS // NW
Q = EMBED // LANES
NB = 5
NROUNDS = U_PER_W // NB
UPAD = BBLK + 8


def _emb_kernel(idx_hbm, tok_hbm, pos_hbm, out_hbm,
                idx_v, g_v, u_v, pos_v, *sems):
    semg = sems[:NB]
    semo = sems[NB:]
    wid = lax.axis_index("s") * NC + lax.axis_index("c")
    u0 = wid * U_PER_W

    pltpu.sync_copy(pos_hbm, pos_v)

    iota = lax.iota(jnp.int32, LANES)
    avecs = [(q * LANES + iota) // 8 for q in range(Q)]
    rvecs = [(q * LANES + iota) % 8 for q in range(Q)]

    def unit_lc(u):
        gu = u0 + u
        return gu // NCBLK, gu % NCBLK

    def idx_gather_start(u, j):
        l, c = unit_lc(u)
        pltpu.sync_copy(idx_hbm.at[pl.ds(l * BATCH + c * BBLK, BBLK)],
                        idx_v.at[j])
        pltpu.async_copy(tok_hbm.at[idx_v.at[j]], g_v.at[j], semg[j])

    def gather_wait(j):
        pltpu.make_async_copy(tok_hbm.at[idx_v.at[j]], g_v.at[j],
                              semg[j]).wait()

    def out_refs(u, j):
        l, c = unit_lc(u)
        return u_v.at[j, :, :, pl.ds(0, BBLK)], out_hbm.at[l, :, c]

    for j in range(NB):
        idx_gather_start(j, j)

    def round_body(r, car):
        for j in range(NB):
            u = r * NB + j
            gather_wait(j)

            @pl.when(r > 0)
            def _(u=u, j=j):
                src, dst = out_refs(u - NB, j)
                pltpu.make_async_copy(src, dst, semo[j]).wait()

            l, c = unit_lc(u)
            pq = [pos_v[l, pl.ds(q * LANES, LANES)] for q in range(Q)]

            @plsc.parallel_loop(0, BBLK, 1, unroll=8)
            def _(t, j=j, pq=pq):
                tvec = jnp.zeros((LANES,), jnp.int32) + t
                for q in range(Q):
                    val = g_v[j, t, pl.ds(q * LANES, LANES)] + pq[q]
                    plsc.store_scatter(u_v.at[j], [avecs[q], rvecs[q], tvec],
                                       val)
            src, dst = out_refs(u, j)
            pltpu.async_copy(src, dst, semo[j])

            @pl.when(r < NROUNDS - 1)
            def _(u=u, j=j):
                idx_gather_start(u + NB, j)
        return car

    lax.fori_loop(0, NROUNDS, round_body, 0)

    for j in range(NB):
        src, dst = out_refs(U_PER_W - NB + j, j)
        pltpu.make_async_copy(src, dst, semo[j]).wait()


@jax.jit
def _run(idx_flat, token_table, pos_table):
    mesh = plsc.VectorSubcoreMesh(core_axis_name="c", subcore_axis_name="s")
    f = pl.kernel(
        _emb_kernel,
        out_type=jax.ShapeDtypeStruct((MAXLEN, 8, NCBLK, 8, BBLK), jnp.float32),
        mesh=mesh,
        scratch_types=[
            pltpu.VMEM((NB, BBLK), jnp.int32),
            pltpu.VMEM((NB, BBLK, EMBED), jnp.float32),
            pltpu.VMEM((NB, 8, 8, UPAD), jnp.float32),
            pltpu.VMEM((MAXLEN, EMBED), jnp.float32),
        ] + [pltpu.SemaphoreType.DMA] * (2 * NB),
        compiler_params=pltpu.CompilerParams(use_tc_tiling_on_sc=False,
                                             needs_layout_passes=False),
    )
    return f(idx_flat, token_table, pos_table)


def kernel(inputs, token_table, pos_table):
    idx_flat = inputs.astype(jnp.int32).T.reshape(-1)
    out5 = _run(idx_flat, token_table, pos_table)
    return out5.transpose(2, 4, 0, 1, 3).reshape(BATCH, MAXLEN, EMBED)


# speedup vs baseline: 2.6646x; 1.0397x over previous
import jax
import jax.numpy as jnp
from jax import lax
from jax.experimental import pallas as pl
from jax.experimental.pallas import tpu as pltpu
from jax.experimental.pallas import tpu_sc as plsc

VOCAB = 100000
MAXLEN = 200
EMBED = 64
BATCH = 1024

NC = 2
NS = 16
NW = NC * NS
LANES = 16

BBLK = 128
NCBLK = BATCH // BBLK
N_UNITS = MAXLEN * NCBLK
U_PER_W = N_UNITS // NW
Q = EMBED // LANES
NB = 5
NROUNDS = U_PER_W // NB
UPAD = BBLK + 8


def _emb_kernel(idx_hbm, tok_hbm, pos_hbm, out_hbm,
                idx_v, g_v, u_v, pos_v, *sems):
    semg = sems[:NB]
    semo = sems[NB:]
    wid = lax.axis_index("s") * NC + lax.axis_index("c")
    u0 = wid * U_PER_W

    pltpu.sync_copy(pos_hbm, pos_v)

    iota = lax.iota(jnp.int32, LANES)
    avecs = [(q * LANES + iota) // 8 for q in range(Q)]
    rvecs = [(q * LANES + iota) % 8 for q in range(Q)]

    def unit_lc(u):
        gu = u0 + u
        return gu // NCBLK, gu % NCBLK

    def idx_gather_start(u, j):
        l, c = unit_lc(u)
        pltpu.sync_copy(idx_hbm.at[l // 8, c, l % 8], idx_v.at[j])
        pltpu.async_copy(tok_hbm.at[idx_v.at[j]], g_v.at[j], semg[j])

    def gather_wait(j):
        pltpu.make_async_copy(tok_hbm.at[idx_v.at[j]], g_v.at[j],
                              semg[j]).wait()

    def out_refs(u, j):
        l, c = unit_lc(u)
        return u_v.at[j, :, :, pl.ds(0, BBLK)], out_hbm.at[l, :, c]

    for j in range(NB):
        idx_gather_start(j, j)

    def round_body(r, car):
        for j in range(NB):
            u = r * NB + j
            gather_wait(j)

            @pl.when(r > 0)
            def _(u=u, j=j):
                src, dst = out_refs(u - NB, j)
                pltpu.make_async_copy(src, dst, semo[j]).wait()

            l, c = unit_lc(u)
            pq = [pos_v[l, pl.ds(q * LANES, LANES)] for q in range(Q)]

            @plsc.parallel_loop(0, BBLK, 1, unroll=16)
            def _(t, j=j, pq=pq):
                tvec = jnp.zeros((LANES,), jnp.int32) + t
                for q in range(Q):
                    val = g_v[j, t, pl.ds(q * LANES, LANES)] + pq[q]
                    plsc.store_scatter(u_v.at[j], [avecs[q], rvecs[q], tvec],
                                       val)
            src, dst = out_refs(u, j)
            pltpu.async_copy(src, dst, semo[j])

            @pl.when(r < NROUNDS - 1)
            def _(u=u, j=j):
                idx_gather_start(u + NB, j)
        return car

    lax.fori_loop(0, NROUNDS, round_body, 0)

    for j in range(NB):
        src, dst = out_refs(U_PER_W - NB + j, j)
        pltpu.make_async_copy(src, dst, semo[j]).wait()


@jax.jit
def _run(idx_flat, token_table, pos_table):
    mesh = plsc.VectorSubcoreMesh(core_axis_name="c", subcore_axis_name="s")
    f = pl.kernel(
        _emb_kernel,
        out_type=jax.ShapeDtypeStruct((MAXLEN, 8, NCBLK, 8, BBLK), jnp.float32),
        mesh=mesh,
        scratch_types=[
            pltpu.VMEM((NB, BBLK), jnp.int32),
            pltpu.VMEM((NB, BBLK, EMBED), jnp.float32),
            pltpu.VMEM((NB, 8, 8, UPAD), jnp.float32),
            pltpu.VMEM((MAXLEN, EMBED), jnp.float32),
        ] + [pltpu.SemaphoreType.DMA] * (2 * NB),
        compiler_params=pltpu.CompilerParams(use_tc_tiling_on_sc=False,
                                             needs_layout_passes=False),
    )
    return f(idx_flat, token_table, pos_table)


def kernel(inputs, token_table, pos_table):
    idx4 = (inputs.astype(jnp.int32).T
            .reshape(MAXLEN // 8, 8, NCBLK, BBLK).swapaxes(1, 2))
    out5 = _run(idx4, token_table, pos_table)
    return out5.transpose(2, 4, 0, 1, 3).reshape(BATCH, MAXLEN, EMBED)
